# baseline (speedup 1.0000x reference)
import jax
import jax.numpy as jnp
from jax.experimental import pallas as pl
from jax.experimental.pallas import tpu as pltpu

_T, _C, _D = 2048, 1024, 64
_BQ = 256
_NB = _T // _BQ


def _attn_kernel(x_ref, wq_ref, wk_ref, wv_ref, bqkv_ref, o_ref,
                 q_scr, k_scr, v_scr):
    i = pl.program_id(0)

    @pl.when(i < _NB)
    def _project():
        xx = x_ref[...].astype(jnp.bfloat16)
        rows = pl.ds(i * _BQ, _BQ)
        q_scr[rows, :] = (
            jnp.dot(xx, wq_ref[...].astype(jnp.bfloat16),
                    preferred_element_type=jnp.float32)
            + bqkv_ref[0, :][None, :]
        ).astype(jnp.bfloat16)
        k_scr[rows, :] = (
            jnp.dot(xx, wk_ref[...].astype(jnp.bfloat16),
                    preferred_element_type=jnp.float32)
            + bqkv_ref[1, :][None, :]
        ).astype(jnp.bfloat16)
        v_scr[rows, :] = (
            jnp.dot(xx, wv_ref[...].astype(jnp.bfloat16),
                    preferred_element_type=jnp.float32)
            + bqkv_ref[2, :][None, :]
        ).astype(jnp.bfloat16)

    @pl.when(i == _NB)
    def _attend():
        scale = 1.0 / (_C ** 0.5)
        mask = (
            jax.lax.broadcasted_iota(jnp.int32, (_BQ, _BQ), 1)
            <= jax.lax.broadcasted_iota(jnp.int32, (_BQ, _BQ), 0)
        )
        for j in range(_NB):
            q_j = q_scr[pl.ds(j * _BQ, _BQ), :]
            k_lo = k_scr[pl.ds(0, (j + 1) * _BQ), :]
            s = jax.lax.dot_general(
                q_j, k_lo, (((1,), (1,)), ((), ())),
                preferred_element_type=jnp.float32,
            ) * scale
            p_diag = jnp.where(mask, jnp.exp(s[:, j * _BQ:]), 0.0)
            if j:
                p = jnp.concatenate([jnp.exp(s[:, : j * _BQ]), p_diag], axis=1)
            else:
                p = p_diag
            denom = jnp.sum(p, axis=1, keepdims=True)
            o = jnp.dot(p.astype(jnp.bfloat16),
                        v_scr[pl.ds(0, (j + 1) * _BQ), :],
                        preferred_element_type=jnp.float32)
            o_ref[pl.ds(j * _BQ, _BQ), :] = o / denom


def kernel(x, Wq, bq, Wk, bk, Wv, bv, gate, mem_keys, mem_vals):
    b, t, c = x.shape
    x2 = x.reshape(t, c)
    bqkv = jnp.stack([bq, bk, bv], axis=0)
    out = pl.pallas_call(
        _attn_kernel,
        grid=(_NB + 1,),
        in_specs=[
            pl.BlockSpec((_BQ, _C), lambda i: (jnp.minimum(i, _NB - 1), 0)),
            pl.BlockSpec((_C, _D), lambda i: (0, 0)),
            pl.BlockSpec((_C, _D), lambda i: (0, 0)),
            pl.BlockSpec((_C, _D), lambda i: (0, 0)),
            pl.BlockSpec((3, _D), lambda i: (0, 0)),
        ],
        out_specs=pl.BlockSpec((_T, _D), lambda i: (0, 0)),
        out_shape=jax.ShapeDtypeStruct((_T, _D), jnp.float32),
        scratch_shapes=[
            pltpu.VMEM((_T, _D), jnp.bfloat16),
            pltpu.VMEM((_T, _D), jnp.bfloat16),
            pltpu.VMEM((_T, _D), jnp.bfloat16),
        ],
        compiler_params=pltpu.CompilerParams(
            dimension_semantics=("arbitrary",),
        ),
    )(x2, Wq, Wk, Wv, bqkv)
    return out.reshape(b, t, _D)


# speedup vs baseline: 1.0941x; 1.0941x over previous
import jax
import jax.numpy as jnp
from jax.experimental import pallas as pl
from jax.experimental.pallas import tpu as pltpu

_T, _C, _D = 2048, 1024, 64
_BQ = 256
_NB = _T // _BQ


def _attn_kernel(x_ref, w_ref, b_ref, o_ref, q_scr, k_scr, v_scr):
    i = pl.program_id(0)

    @pl.when(i < _NB)
    def _project():
        xx = x_ref[...].astype(jnp.bfloat16)
        rows = pl.ds(i * _BQ, _BQ)
        qkv = (
            jnp.dot(xx, w_ref[...], preferred_element_type=jnp.float32)
            + b_ref[...]
        ).astype(jnp.bfloat16)
        q_scr[rows, :] = qkv[:, :_D]
        k_scr[rows, :] = qkv[:, _D:2 * _D]
        v_scr[rows, :] = jnp.concatenate(
            [qkv[:, 2 * _D:],
             jnp.ones((_BQ, 1), jnp.bfloat16),
             jnp.zeros((_BQ, _D - 1), jnp.bfloat16)], axis=1)

    @pl.when(i == _NB)
    def _attend():
        mask = (
            jax.lax.broadcasted_iota(jnp.int32, (_BQ, _BQ), 1)
            <= jax.lax.broadcasted_iota(jnp.int32, (_BQ, _BQ), 0)
        )
        for j in range(_NB):
            q_j = q_scr[pl.ds(j * _BQ, _BQ), :]
            k_lo = k_scr[pl.ds(0, (j + 1) * _BQ), :]
            s = jax.lax.dot_general(
                q_j, k_lo, (((1,), (1,)), ((), ())),
                preferred_element_type=jnp.float32,
            ).astype(jnp.bfloat16)
            p_diag = jnp.where(mask, jnp.exp(s[:, j * _BQ:]),
                               jnp.bfloat16(0.0))
            if j:
                p = jnp.concatenate([jnp.exp(s[:, : j * _BQ]), p_diag],
                                    axis=1)
            else:
                p = p_diag
            o = jnp.dot(p, v_scr[pl.ds(0, (j + 1) * _BQ), :],
                        preferred_element_type=jnp.float32)
            denom = o[:, _D:_D + 1]
            o_ref[pl.ds(j * _BQ, _BQ), :] = o[:, :_D] * (1.0 / denom)


def kernel(x, Wq, bq, Wk, bk, Wv, bv, gate, mem_keys, mem_vals):
    b, t, c = x.shape
    x2 = x.reshape(t, c)
    scale = 1.0 / (c ** 0.5)
    w = jnp.concatenate([Wq * scale, Wk, Wv], axis=1).astype(jnp.bfloat16)
    bias = jnp.concatenate([bq * scale, bk, bv])[None, :]
    out = pl.pallas_call(
        _attn_kernel,
        grid=(_NB + 1,),
        in_specs=[
            pl.BlockSpec((_BQ, _C), lambda i: (jnp.minimum(i, _NB - 1), 0)),
            pl.BlockSpec((_C, 3 * _D), lambda i: (0, 0)),
            pl.BlockSpec((1, 3 * _D), lambda i: (0, 0)),
        ],
        out_specs=pl.BlockSpec((_T, _D), lambda i: (0, 0)),
        out_shape=jax.ShapeDtypeStruct((_T, _D), jnp.float32),
        scratch_shapes=[
            pltpu.VMEM((_T, _D), jnp.bfloat16),
            pltpu.VMEM((_T, _D), jnp.bfloat16),
            pltpu.VMEM((_T, 2 * _D), jnp.bfloat16),
        ],
        compiler_params=pltpu.CompilerParams(
            dimension_semantics=("arbitrary",),
        ),
    )(x2, w, bias)
    return out.reshape(b, t, _D)
